# baseline (speedup 1.0000x reference)
import jax
import jax.numpy as jnp
from jax import lax
from jax.experimental import pallas as pl
from jax.experimental.pallas import tpu as pltpu
from jax.experimental.pallas import tpu_sc as plsc

_DIM = 128
_B = 4096
_W = 20
_K = 50
_NC = 2
_NS = 16
_L = 16
_NW = _NC * _NS
_BPW = _B // _NW
_CB = 2
_NCHUNK = _BPW // _CB
_YC = _CB * _W
_KC = _CB * _K
_NSEG = _DIM // _L
_NEG_SCALE = 1.0 / (_B * _K)


def _log_sigmoid(v):
    u = jnp.exp(-jnp.abs(v))
    s = u / (u + 2.0)
    s2 = s * s
    poly = 1.0 + s2 * (
        (1.0 / 3.0)
        + s2 * ((1.0 / 5.0) + s2 * ((1.0 / 7.0) + s2 * ((1.0 / 9.0) + s2 * (1.0 / 11.0))))
    )
    return jnp.minimum(v, 0.0) - 2.0 * s * poly


def _iota():
    return lax.iota(jnp.int32, _L)


def _tree_sum(terms):
    ts = list(terms)
    while len(ts) > 1:
        ts = [ts[i] + ts[i + 1] for i in range(0, len(ts) - 1, 2)] + (
            [ts[-1]] if len(ts) % 2 else [])
    return ts[0]


def _perm_xor(v, b):
    return jnp.take_along_axis(v, _iota() ^ b, axis=0, mode="promise_in_bounds")


def _group_dots(buf, xv, rows_per_elem, g, nval):
    stack = []
    for r16 in range(nval):
        row = g * _L + r16
        e = row // rows_per_elem
        v = _tree_sum([buf[row, pl.ds(_L * j, _L)] * xv[e][j]
                       for j in range(_NSEG)])
        lvl = 0
        while stack and stack[-1][0] == lvl:
            prev = stack.pop()[1]
            b = 1 << lvl
            v = jnp.where((_iota() & b) == 0,
                          prev + _perm_xor(prev, b),
                          v + _perm_xor(v, b))
            lvl += 1
        stack.append((lvl, v))
    d = stack[0][1]
    b = nval
    while b < _L:
        d = d + _perm_xor(d, b)
        b *= 2
    return d


def _sgns_body(bx_hbm, by_hbm, bn_hbm, in_emb, out_emb, out_hbm,
               bxv, byv, bnv, xall, ybuf, nbuf, accp,
               semx, semy0, semn0, semy1, semn1):
    wid = lax.axis_index("s") * _NC + lax.axis_index("c")
    base = wid * _BPW

    pltpu.sync_copy(bx_hbm.at[pl.ds(base, _BPW)], bxv)
    pltpu.sync_copy(by_hbm.at[pl.ds(base * _W, _BPW * _W)], byv.at[pl.ds(0, _BPW * _W)])
    pltpu.sync_copy(bn_hbm.at[pl.ds(wid * _NCHUNK, _NCHUNK)], bnv)
    pltpu.async_copy(in_emb.at[bxv], xall, semx).wait()

    sems = ((semy0, semn0), (semy1, semn1))

    def issue(c, buf):
        sy, sn = sems[buf]
        pltpu.async_copy(out_emb.at[byv.at[pl.ds(c * _YC, _YC)]],
                         ybuf.at[buf], sy)
        pltpu.async_copy(out_emb.at[bnv.at[c]], nbuf.at[buf], sn)

    def drain(c, buf):
        sy, sn = sems[buf]
        pltpu.make_async_copy(out_emb.at[byv.at[pl.ds(c * _YC, _YC)]],
                              ybuf.at[buf], sy).wait()
        pltpu.make_async_copy(out_emb.at[bnv.at[c]], nbuf.at[buf], sn).wait()

    def compute(c, buf, ay, an):
        xv = [[xall[c * _CB + e, pl.ds(_L * j, _L)] for j in range(_NSEG)]
              for e in range(_CB)]

        for g in range((_YC + _L - 1) // _L):
            nval = min(_L, _YC - g * _L)
            d = _group_dots(ybuf.at[buf], xv, _W, g, nval)
            mvec = byv[pl.ds(c * _YC + g * _L, _L)]
            ok = mvec != 0
            if nval < _L:
                ok = ok & (_iota() < nval)
            ay = ay + jnp.where(ok, _log_sigmoid(d), 0.0)

        for g in range((_KC + _L - 1) // _L):
            nval = min(_L, _KC - g * _L)
            d = _group_dots(nbuf.at[buf], xv, _K, g, nval)
            val = _log_sigmoid(-d)
            if nval < _L:
                val = jnp.where(_iota() < nval, val, 0.0)
            an = an + val
        return ay, an

    issue(0, 0)
    zero16 = jnp.zeros((_L,), jnp.float32)

    def pair(i, carry):
        ay, an = carry
        issue(2 * i + 1, 1)
        drain(2 * i, 0)
        ay, an = compute(2 * i, 0, ay, an)

        @pl.when(i < _NCHUNK // 2 - 1)
        def _():
            issue(2 * i + 2, 0)

        drain(2 * i + 1, 1)
        ay, an = compute(2 * i + 1, 1, ay, an)
        return ay, an

    acc_y, acc_n = lax.fori_loop(0, _NCHUNK // 2, pair, (zero16, zero16))

    accp[...] = acc_y + acc_n * jnp.float32(_NEG_SCALE)
    pltpu.sync_copy(accp, out_hbm.at[wid])


@jax.jit
def _sgns_partials(batch_X, by_flat, bn2, in_embedding, out_embedding):
    mesh = plsc.VectorSubcoreMesh(core_axis_name="c", subcore_axis_name="s")
    return pl.kernel(
        _sgns_body,
        out_type=jax.ShapeDtypeStruct((_NW, _L), jnp.float32),
        mesh=mesh,
        compiler_params=pltpu.CompilerParams(needs_layout_passes=False),
        scratch_types=[
            pltpu.VMEM((_BPW,), jnp.int32),
            pltpu.VMEM((_BPW * _W + _L,), jnp.int32),
            pltpu.VMEM((_NCHUNK, _KC), jnp.int32),
            pltpu.VMEM((_BPW, _DIM), jnp.float32),
            pltpu.VMEM((2, _YC, _DIM), jnp.float32),
            pltpu.VMEM((2, _KC, _DIM), jnp.float32),
            pltpu.VMEM((_L,), jnp.float32),
            pltpu.SemaphoreType.DMA,
            pltpu.SemaphoreType.DMA,
            pltpu.SemaphoreType.DMA,
            pltpu.SemaphoreType.DMA,
            pltpu.SemaphoreType.DMA,
        ],
    )(batch_X, by_flat, bn2, in_embedding, out_embedding)


def kernel(batch_X, batch_y, batch_N, in_embedding, out_embedding):
    by_flat = batch_y.reshape(_B * _W)
    bn2 = batch_N.reshape(_B * _K // _KC, _KC)
    parts = _sgns_partials(batch_X, by_flat, bn2, in_embedding, out_embedding)
    return -jnp.sum(parts)


# speedup vs baseline: 1.2566x; 1.2566x over previous
import jax
import jax.numpy as jnp
from jax import lax
from jax.experimental import pallas as pl
from jax.experimental.pallas import tpu as pltpu
from jax.experimental.pallas import tpu_sc as plsc

_DIM = 128
_B = 4096
_W = 20
_WP = 32
_K = 50
_NC = 2
_NS = 16
_L = 16
_NW = _NC * _NS
_BPW = _B // _NW
_NSEG = _DIM // _L
_NEG_SCALE = 1.0 / (_B * _K)


def _iota():
    return lax.iota(jnp.int32, _L)


def _log_sigmoid(v):
    u = jnp.exp(-jnp.abs(v))
    s = u / (u + 2.0)
    s2 = s * s
    poly = 1.0 + s2 * (
        (1.0 / 3.0)
        + s2 * ((1.0 / 5.0) + s2 * ((1.0 / 7.0) + s2 * ((1.0 / 9.0) + s2 * (1.0 / 11.0))))
    )
    return jnp.minimum(v, 0.0) - 2.0 * s * poly


def _tree_sum(terms):
    ts = list(terms)
    while len(ts) > 1:
        ts = [ts[i] + ts[i + 1] for i in range(0, len(ts) - 1, 2)] + (
            [ts[-1]] if len(ts) % 2 else [])
    return ts[0]


def _perm_xor(v, b):
    return jnp.take_along_axis(v, _iota() ^ b, axis=0, mode="promise_in_bounds")


def _group_dots(buf, xv, row0, nval):
    stack = []
    for r16 in range(nval):
        row = row0 + r16
        v = _tree_sum([buf[row, pl.ds(_L * j, _L)] * xv[j]
                       for j in range(_NSEG)])
        lvl = 0
        while stack and stack[-1][0] == lvl:
            prev = stack.pop()[1]
            b = 1 << lvl
            v = jnp.where((_iota() & b) == 0,
                          prev + _perm_xor(prev, b),
                          v + _perm_xor(v, b))
            lvl += 1
        stack.append((lvl, v))
    d = stack[0][1]
    b = nval
    while b < _L:
        d = d + _perm_xor(d, b)
        b *= 2
    return d


def _sgns_body(bx_hbm, by_hbm, bn_hbm, in_emb, out_emb, out_hbm,
               bxv, byv, bnv, xall, ybuf, nbuf, accp,
               semx, semy0, semn0, semy1, semn1):
    wid = lax.axis_index("s") * _NC + lax.axis_index("c")
    base = wid * _BPW

    pltpu.sync_copy(bx_hbm.at[pl.ds(base, _BPW)], bxv)
    pltpu.sync_copy(by_hbm.at[pl.ds(base, _BPW)], byv)
    pltpu.sync_copy(bn_hbm.at[pl.ds(base, _BPW)], bnv)
    pltpu.async_copy(in_emb.at[bxv], xall, semx).wait()

    sems = ((semy0, semn0), (semy1, semn1))

    def issue(c, buf):
        sy, sn = sems[buf]
        pltpu.async_copy(out_emb.at[byv.at[c, pl.ds(0, _W)]], ybuf.at[buf], sy)
        pltpu.async_copy(out_emb.at[bnv.at[c]], nbuf.at[buf], sn)

    def drain(c, buf):
        sy, sn = sems[buf]
        pltpu.make_async_copy(out_emb.at[byv.at[c, pl.ds(0, _W)]],
                              ybuf.at[buf], sy).wait()
        pltpu.make_async_copy(out_emb.at[bnv.at[c]], nbuf.at[buf], sn).wait()

    def compute(c, buf, ay, an):
        xv = [xall[c, pl.ds(_L * j, _L)] for j in range(_NSEG)]

        d = _group_dots(ybuf.at[buf], xv, 0, _L)
        ok = byv[c, pl.ds(0, _L)] != 0
        ay = ay + jnp.where(ok, _log_sigmoid(d), 0.0)

        d = _group_dots(ybuf.at[buf], xv, _L, _W - _L)
        ok = byv[c, pl.ds(_L, _L)] != 0
        ay = ay + jnp.where(ok, _log_sigmoid(d), 0.0)

        for row0 in range(0, _K - _L + 1, _L):
            d = _group_dots(nbuf.at[buf], xv, row0, _L)
            an = an + _log_sigmoid(-d)
        d = _group_dots(nbuf.at[buf], xv, (_K // _L) * _L, _K % _L)
        an = an + jnp.where(_iota() < _K % _L, _log_sigmoid(-d), 0.0)
        return ay, an

    issue(0, 0)
    zero16 = jnp.zeros((_L,), jnp.float32)

    def pair(i, carry):
        ay, an = carry
        issue(2 * i + 1, 1)
        drain(2 * i, 0)
        ay, an = compute(2 * i, 0, ay, an)

        @pl.when(i < _BPW // 2 - 1)
        def _():
            issue(2 * i + 2, 0)

        drain(2 * i + 1, 1)
        ay, an = compute(2 * i + 1, 1, ay, an)
        return ay, an

    acc_y, acc_n = lax.fori_loop(0, _BPW // 2, pair, (zero16, zero16))

    accp[...] = acc_y + acc_n * jnp.float32(_NEG_SCALE)
    pltpu.sync_copy(accp, out_hbm.at[wid])


@jax.jit
def _sgns_partials(batch_X, by_pad, batch_N, in_embedding, out_embedding):
    mesh = plsc.VectorSubcoreMesh(core_axis_name="c", subcore_axis_name="s")
    return pl.kernel(
        _sgns_body,
        out_type=jax.ShapeDtypeStruct((_NW, _L), jnp.float32),
        mesh=mesh,
        compiler_params=pltpu.CompilerParams(needs_layout_passes=False),
        scratch_types=[
            pltpu.VMEM((_BPW,), jnp.int32),
            pltpu.VMEM((_BPW, _WP), jnp.int32),
            pltpu.VMEM((_BPW, _K), jnp.int32),
            pltpu.VMEM((_BPW, _DIM), jnp.float32),
            pltpu.VMEM((2, _W, _DIM), jnp.float32),
            pltpu.VMEM((2, _K, _DIM), jnp.float32),
            pltpu.VMEM((_L,), jnp.float32),
            pltpu.SemaphoreType.DMA,
            pltpu.SemaphoreType.DMA,
            pltpu.SemaphoreType.DMA,
            pltpu.SemaphoreType.DMA,
            pltpu.SemaphoreType.DMA,
        ],
    )(batch_X, by_pad, batch_N, in_embedding, out_embedding)


def kernel(batch_X, batch_y, batch_N, in_embedding, out_embedding):
    by_pad = jnp.pad(batch_y, ((0, 0), (0, _WP - _W)))
    parts = _sgns_partials(batch_X, by_pad, batch_N, in_embedding, out_embedding)
    return -jnp.sum(parts)
